# baseline (speedup 1.0000x reference)
import functools

import jax
import jax.numpy as jnp
from jax.experimental import pallas as pl
from jax.experimental.pallas import tpu as pltpu
from jax.experimental.pallas import tpu_sc as plsc

B, S, D = 2, 2048, 768
NF, DFE, DE, DR, E, K = 32, 32, 128, 128, 8, 2
DIN = D + DFE
BLK = 512
NBLK = (B * S) // BLK
BLKS_PER_BATCH = S // BLK


def _moe_body(seq_ref, hidden_ref, feat_ref,
              w_feat_ref, b_feat_ref,
              w_r1_ref, b_r1_ref, w_r2_ref, b_r2_ref,
              w_e1_ref, b_e1_ref, w_e2_ref, b_e2_ref,
              next_ref, delta_ref, gate_w_ref, gate_l_ref, aux_ref,
              ah_scr, af_scr, b2_scr):
    i = pl.program_id(0)
    batch = i // BLKS_PER_BATCH
    pos0 = (i % BLKS_PER_BATCH) * BLK
    seq_len = seq_ref[batch]
    bf = jnp.bfloat16

    @pl.when(i == 0)
    def _():
        for e in range(E):
            blk = w_e1_ref[pl.ds(e * DIN, DIN), :].astype(bf)
            ah_scr[:, pl.ds(e * DE, DE)] = blk[:D, :]
            af_scr[:, pl.ds(e * DE, DE)] = blk[D:, :]
        b2_scr[...] = w_e2_ref[...].astype(bf)

    hidden = hidden_ref[...]
    feat = feat_ref[...]

    f = jax.nn.gelu(feat @ w_feat_ref[...] + b_feat_ref[...])

    r = jax.nn.gelu(hidden @ w_r1_ref[:D, :] + f @ w_r1_ref[D:, :]
                    + b_r1_ref[...])
    gl = jax.lax.dot_general(w_r2_ref[...], r, (((0,), (1,)), ((), ())),
                             preferred_element_type=jnp.float32)
    gl = gl + b_r2_ref[...]
    gate_l_ref[...] = gl

    iota_e = jax.lax.broadcasted_iota(jnp.int32, (E, BLK), 0)
    m1 = jnp.max(gl, axis=0, keepdims=True)
    i1 = jnp.min(jnp.where(gl == m1, iota_e, E), axis=0, keepdims=True)
    neg = jnp.finfo(jnp.float32).min
    masked = jnp.where(iota_e == i1, neg, gl)
    m2 = jnp.max(masked, axis=0, keepdims=True)
    i2 = jnp.min(jnp.where(masked == m2, iota_e, E), axis=0, keepdims=True)
    e21 = jnp.exp(m2 - m1)
    w1 = 1.0 / (1.0 + e21)
    w2 = 1.0 - w1
    gate_w = (jnp.where(iota_e == i1, w1, 0.0)
              + jnp.where(iota_e == i2, w2, 0.0))
    gate_w_ref[...] = gate_w

    @pl.when(i == 0)
    def _():
        aux_ref[...] = jnp.zeros_like(aux_ref)
    aux_ref[...] += jnp.sum(gate_w, axis=1, keepdims=True) * (1.0 / (B * S))

    iota_t = jax.lax.broadcasted_iota(jnp.int32, (1, BLK), 1) + pos0
    tok_mask = (iota_t < seq_len).astype(jnp.float32)

    @pl.when(pos0 < seq_len)
    def _():
        gw_eff = gate_w * tok_mask
        h1 = jnp.dot(hidden.astype(bf), ah_scr[...],
                     preferred_element_type=jnp.float32)
        h2 = jnp.dot(f.astype(bf), af_scr[...],
                     preferred_element_type=jnp.float32)
        h = jax.nn.gelu(h1 + h2 + b_e1_ref[...])
        expand = (jax.lax.broadcasted_iota(jnp.int32, (E, E * DE), 1) // DE
                  == jax.lax.broadcasted_iota(jnp.int32, (E, E * DE), 0)
                  ).astype(jnp.float32)
        ge = jax.lax.dot_general(gw_eff, expand, (((0,), (0,)), ((), ())),
                                 preferred_element_type=jnp.float32)
        hg = (h * ge).astype(bf)
        comb = (jnp.dot(hg, b2_scr[...], preferred_element_type=jnp.float32)
                + jax.lax.dot_general(gw_eff, b_e2_ref[...],
                                      (((0,), (0,)), ((), ())),
                                      preferred_element_type=jnp.float32))
        delta_ref[...] = comb
        next_ref[...] = hidden + comb

    @pl.when(pos0 >= seq_len)
    def _():
        delta_ref[...] = jnp.zeros_like(delta_ref)
        next_ref[...] = hidden


_NW = 32
_TCH = (B * S) // (_NW // E)
_JCH = 128


def _gate_tr_body(gw_hbm, gl_hbm, idx_hbm, gw_out, gl_out,
                  seg_v, idx_v):
    wid = jax.lax.axis_index("s") * 2 + jax.lax.axis_index("c")
    e = wid % E
    t0 = (wid // E) * _TCH
    for src_hbm, dst_hbm in ((gw_hbm, gw_out), (gl_hbm, gl_out)):
        pltpu.sync_copy(src_hbm.at[e, pl.ds(t0, _TCH)], seg_v)
        for j in range(_TCH // _JCH):
            pltpu.sync_copy(idx_hbm.at[e, pl.ds(t0 + j * _JCH, _JCH)], idx_v)
            pltpu.sync_copy(seg_v.at[pl.ds(j * _JCH, _JCH)],
                            dst_hbm.at[idx_v])


def _gate_relayout(gw_t, gl_t):
    idx = (jnp.arange(B * S, dtype=jnp.int32)[None, :] * E
           + jnp.arange(E, dtype=jnp.int32)[:, None])
    mesh = plsc.VectorSubcoreMesh(core_axis_name="c", subcore_axis_name="s")
    k = functools.partial(
        pl.kernel,
        mesh=mesh,
        out_type=[
            jax.ShapeDtypeStruct((B * S * E,), jnp.float32),
            jax.ShapeDtypeStruct((B * S * E,), jnp.float32),
        ],
        scratch_types=[
            pltpu.VMEM((_TCH,), jnp.float32),
            pltpu.VMEM((_JCH,), jnp.int32),
        ],
    )(_gate_tr_body)
    return k(gw_t, gl_t, idx)


@functools.partial(jax.jit, static_argnames=())
def kernel(hidden, feat, item_seq_len, W_feat, b_feat, W_r1, b_r1, W_r2,
           b_r2, W_e1, b_e1, W_e2, b_e2):
    hidden2 = hidden.reshape(B * S, D)
    feat2 = feat.reshape(B * S, NF)
    seq = item_seq_len.astype(jnp.int32)

    full = lambda shape: pl.BlockSpec(shape, lambda i: (0,) * len(shape))

    out = pl.pallas_call(
        _moe_body,
        grid=(NBLK,),
        in_specs=[
            pl.BlockSpec(memory_space=pltpu.SMEM),
            pl.BlockSpec((BLK, D), lambda i: (i, 0)),
            pl.BlockSpec((BLK, NF), lambda i: (i, 0)),
            full((NF, DFE)), full((DFE,)),
            full((DIN, DR)), full((DR,)),
            full((DR, E)), full((E, 1)),
            full((E * DIN, DE)), full((1, E * DE)),
            full((E * DE, D)), full((E, D)),
        ],
        out_specs=[
            pl.BlockSpec((BLK, D), lambda i: (i, 0)),
            pl.BlockSpec((BLK, D), lambda i: (i, 0)),
            pl.BlockSpec((E, BLK), lambda i: (0, i)),
            pl.BlockSpec((E, BLK), lambda i: (0, i)),
            pl.BlockSpec((E, 1), lambda i: (0, 0)),
        ],
        out_shape=[
            jax.ShapeDtypeStruct((B * S, D), jnp.float32),
            jax.ShapeDtypeStruct((B * S, D), jnp.float32),
            jax.ShapeDtypeStruct((E, B * S), jnp.float32),
            jax.ShapeDtypeStruct((E, B * S), jnp.float32),
            jax.ShapeDtypeStruct((E, 1), jnp.float32),
        ],
        scratch_shapes=[
            pltpu.VMEM((D, E * DE), jnp.bfloat16),
            pltpu.VMEM((DFE, E * DE), jnp.bfloat16),
            pltpu.VMEM((E * DE, D), jnp.bfloat16),
        ],
    )(seq, hidden2, feat2, W_feat, b_feat,
      W_r1, b_r1, W_r2, b_r2.reshape(E, 1),
      W_e1.reshape(E * DIN, DE), b_e1.reshape(1, E * DE),
      W_e2.reshape(E * DE, D), b_e2)

    next_h, delta, gate_w_t, gate_l_t, aux = out
    gate_w_f, gate_l_f = _gate_relayout(gate_w_t, gate_l_t)
    return (next_h.reshape(B, S, D), delta.reshape(B, S, D),
            gate_w_f.reshape(B, S, E), gate_l_f.reshape(B, S, E),
            aux.reshape(E))


# speedup vs baseline: 6.8735x; 6.8735x over previous
import functools

import jax
import jax.numpy as jnp
from jax.experimental import pallas as pl
from jax.experimental.pallas import tpu as pltpu

B, S, D = 2, 2048, 768
NF, DFE, DE, DR, E, K = 32, 32, 128, 128, 8, 2
DIN = D + DFE
BLK = 512
NBLK = (B * S) // BLK
BLKS_PER_BATCH = S // BLK


def _moe_body(seq_ref, hidden_ref, feat_ref,
              w_feat_ref, b_feat_ref,
              w_r1_ref, b_r1_ref, w_r2_ref, b_r2_ref,
              w_e1_ref, b_e1_ref, w_e2_ref, b_e2_ref,
              next_ref, delta_ref, gate_w_ref, gate_l_ref, aux_ref,
              ah_scr, af_scr, b2_scr):
    i = pl.program_id(0)
    batch = i // BLKS_PER_BATCH
    pos0 = (i % BLKS_PER_BATCH) * BLK
    seq_len = seq_ref[batch]
    bf = jnp.bfloat16

    @pl.when(i == 0)
    def _():
        for e in range(E):
            blk = w_e1_ref[pl.ds(e * DIN, DIN), :].astype(bf)
            ah_scr[:, pl.ds(e * DE, DE)] = blk[:D, :]
            af_scr[:, pl.ds(e * DE, DE)] = blk[D:, :]
        b2_scr[...] = w_e2_ref[...].astype(bf)

    hidden = hidden_ref[...]
    feat = feat_ref[...]

    f = jax.nn.gelu(feat @ w_feat_ref[...] + b_feat_ref[...])

    r = jax.nn.gelu(hidden @ w_r1_ref[:D, :] + f @ w_r1_ref[D:, :]
                    + b_r1_ref[...])
    gl = jax.lax.dot_general(w_r2_ref[...], r, (((0,), (1,)), ((), ())),
                             preferred_element_type=jnp.float32)
    gl = gl + b_r2_ref[...]
    gate_l_ref[...] = gl

    iota_e = jax.lax.broadcasted_iota(jnp.int32, (E, BLK), 0)
    m1 = jnp.max(gl, axis=0, keepdims=True)
    i1 = jnp.min(jnp.where(gl == m1, iota_e, E), axis=0, keepdims=True)
    neg = jnp.finfo(jnp.float32).min
    masked = jnp.where(iota_e == i1, neg, gl)
    m2 = jnp.max(masked, axis=0, keepdims=True)
    i2 = jnp.min(jnp.where(masked == m2, iota_e, E), axis=0, keepdims=True)
    e21 = jnp.exp(m2 - m1)
    w1 = 1.0 / (1.0 + e21)
    w2 = 1.0 - w1
    gate_w = (jnp.where(iota_e == i1, w1, 0.0)
              + jnp.where(iota_e == i2, w2, 0.0))
    gate_w_ref[...] = gate_w

    @pl.when(i == 0)
    def _():
        aux_ref[...] = jnp.zeros_like(aux_ref)
    aux_ref[...] += jnp.sum(gate_w, axis=1, keepdims=True) * (1.0 / (B * S))

    iota_t = jax.lax.broadcasted_iota(jnp.int32, (1, BLK), 1) + pos0
    tok_mask = (iota_t < seq_len).astype(jnp.float32)

    @pl.when(pos0 < seq_len)
    def _():
        gw_eff = gate_w * tok_mask
        h1 = jnp.dot(hidden.astype(bf), ah_scr[...],
                     preferred_element_type=jnp.float32)
        h2 = jnp.dot(f.astype(bf), af_scr[...],
                     preferred_element_type=jnp.float32)
        h = jax.nn.gelu(h1 + h2 + b_e1_ref[...])
        expand = (jax.lax.broadcasted_iota(jnp.int32, (E, E * DE), 1) // DE
                  == jax.lax.broadcasted_iota(jnp.int32, (E, E * DE), 0)
                  ).astype(jnp.float32)
        ge = jax.lax.dot_general(gw_eff, expand, (((0,), (0,)), ((), ())),
                                 preferred_element_type=jnp.float32)
        hg = (h * ge).astype(bf)
        comb = (jnp.dot(hg, b2_scr[...], preferred_element_type=jnp.float32)
                + jax.lax.dot_general(gw_eff, b_e2_ref[...],
                                      (((0,), (0,)), ((), ())),
                                      preferred_element_type=jnp.float32))
        delta_ref[...] = comb
        next_ref[...] = hidden + comb

    @pl.when(pos0 >= seq_len)
    def _():
        delta_ref[...] = jnp.zeros_like(delta_ref)
        next_ref[...] = hidden


@functools.partial(jax.jit, static_argnames=())
def kernel(hidden, feat, item_seq_len, W_feat, b_feat, W_r1, b_r1, W_r2,
           b_r2, W_e1, b_e1, W_e2, b_e2):
    hidden2 = hidden.reshape(B * S, D)
    feat2 = feat.reshape(B * S, NF)
    seq = item_seq_len.astype(jnp.int32)

    full = lambda shape: pl.BlockSpec(shape, lambda i: (0,) * len(shape))

    out = pl.pallas_call(
        _moe_body,
        grid=(NBLK,),
        in_specs=[
            pl.BlockSpec(memory_space=pltpu.SMEM),
            pl.BlockSpec((BLK, D), lambda i: (i, 0)),
            pl.BlockSpec((BLK, NF), lambda i: (i, 0)),
            full((NF, DFE)), full((DFE,)),
            full((DIN, DR)), full((DR,)),
            full((DR, E)), full((E, 1)),
            full((E * DIN, DE)), full((1, E * DE)),
            full((E * DE, D)), full((E, D)),
        ],
        out_specs=[
            pl.BlockSpec((BLK, D), lambda i: (i, 0)),
            pl.BlockSpec((BLK, D), lambda i: (i, 0)),
            pl.BlockSpec((E, BLK), lambda i: (0, i)),
            pl.BlockSpec((E, BLK), lambda i: (0, i)),
            pl.BlockSpec((E, 1), lambda i: (0, 0)),
        ],
        out_shape=[
            jax.ShapeDtypeStruct((B * S, D), jnp.float32),
            jax.ShapeDtypeStruct((B * S, D), jnp.float32),
            jax.ShapeDtypeStruct((E, B * S), jnp.float32),
            jax.ShapeDtypeStruct((E, B * S), jnp.float32),
            jax.ShapeDtypeStruct((E, 1), jnp.float32),
        ],
        scratch_shapes=[
            pltpu.VMEM((D, E * DE), jnp.bfloat16),
            pltpu.VMEM((DFE, E * DE), jnp.bfloat16),
            pltpu.VMEM((E * DE, D), jnp.bfloat16),
        ],
    )(seq, hidden2, feat2, W_feat, b_feat,
      W_r1, b_r1, W_r2, b_r2.reshape(E, 1),
      W_e1.reshape(E * DIN, DE), b_e1.reshape(1, E * DE),
      W_e2.reshape(E * DE, D), b_e2)

    next_h, delta, gate_w_t, gate_l_t, aux = out
    return (next_h.reshape(B, S, D), delta.reshape(B, S, D),
            gate_w_t.T.reshape(B, S, E), gate_l_t.T.reshape(B, S, E),
            aux.reshape(E))
